# baseline (speedup 1.0000x reference)
import functools

import jax
import jax.numpy as jnp
from jax import lax
from jax.experimental import pallas as pl
from jax.experimental.pallas import tpu as pltpu
from jax.experimental.pallas import tpu_sc as plsc

B = 16
NCLS = 80
C = NCLS + 5
N = 20
NROW = 85
NW = 32


def _softplus16(v):
    m = jnp.maximum(v, 0.0)
    t = jnp.exp(-jnp.abs(v))
    z = t / (2.0 + t)
    z2 = z * z
    p = 1.0 / 9.0
    p = 1.0 / 7.0 + z2 * p
    p = 1.0 / 5.0 + z2 * p
    p = 1.0 / 3.0 + z2 * p
    p = 1.0 + z2 * p
    return m + 2.0 * z * p


def _compute_cells(tb_v, flat_v, H, W, k0):
    lanes = lax.iota(jnp.int32, 16)
    out = []
    for k in range(2):
        n = lanes + 16 * k
        nv = jnp.minimum(n, N - 1)
        i4 = nv * 4
        x = plsc.load_gather(tb_v, [i4])
        y = plsc.load_gather(tb_v, [i4 + 1])
        gx = jnp.clip((x * float(W)).astype(jnp.int32), 0, W - 1)
        gy = jnp.clip((y * float(H)).astype(jnp.int32), 0, H - 1)
        fl = gy * W + gx
        fl = jnp.where(n < N, fl, -1)
        flat_v[pl.ds(k0 + 16 * k, 16)] = fl
        out.append(fl)
    return out


def _fire_cells(det, g_v, gsem, fls, b, W, r0):
    lanes = lax.iota(jnp.int32, 16)
    shift = W.bit_length() - 1
    for n in range(N):
        fl = fls[n // 16]
        fl_s = jnp.sum(jnp.where(lanes == (n % 16), fl, 0))
        gy = lax.shift_right_logical(fl_s, shift)
        gx = fl_s & (W - 1)
        pltpu.make_async_copy(
            det.at[b, gy, pl.ds(gx, 1), :],
            g_v.at[pl.ds(r0 + n, 1), :],
            gsem).start()


def _dense_scan(det, vb0, vb1, dsem0, dsem1, H, W, w, nper):
    nslab = H // 4
    lanes = lax.iota(jnp.int32, 16)
    bufs = (vb0, vb1)
    sems = (dsem0, dsem1)

    def fire(j, buf, sem):
        u = w * nper + j
        bb = u // nslab
        y0 = (u % nslab) * 4
        pltpu.make_async_copy(det.at[bb, pl.ds(y0, 4), :, :],
                              buf.at[:, pl.ds(0, W), :], sem).start()

    def wait(buf, sem):
        pltpu.make_async_copy(det.at[0, pl.ds(0, 4), :, :],
                              buf.at[:, pl.ds(0, W), :], sem).wait()

    def extract(buf):
        acc = jnp.zeros((16,), jnp.float32)
        c4 = jnp.full((16,), 4, jnp.int32)
        shift = W.bit_length() - 1
        nvec = (4 * W) // 16
        for i in range(nvec):
            cell = lanes + 16 * i
            r = lax.shift_right_logical(cell, shift)
            x = cell & (W - 1)
            acc = acc + _softplus16(plsc.load_gather(buf, [r, x, c4]))
        return jnp.sum(acc)

    fire(0, bufs[0], sems[0])
    s = jnp.float32(0.0)
    for j in range(nper):
        if j + 1 < nper:
            fire(j + 1, bufs[(j + 1) % 2], sems[(j + 1) % 2])
        wait(bufs[j % 2], sems[j % 2])
        s = s + extract(bufs[j % 2])
    return s


def _sparse_task(g_v, tb_v, tc_v, flat_v, keep_v, pk_v, r0, k0):
    lanes = lax.iota(jnp.int32, 16)

    f0 = flat_v[pl.ds(k0, 16)]
    f1 = flat_v[pl.ds(k0 + 16, 16)]
    c0 = tc_v[pl.ds(0, 16)]
    c1 = tc_v[pl.ds(16, 16)]

    def dbody(m, carry):
        d0, d1, p0, p1 = carry
        fm = plsc.load_gather(flat_v, [jnp.full((16,), k0 + m, jnp.int32)])
        cm = plsc.load_gather(tc_v, [jnp.full((16,), m, jnp.int32)])
        e0 = (f0 == fm) & (lanes < m)
        e1 = (f1 == fm) & ((lanes + 16) < m)
        d0 = d0 | e0
        d1 = d1 | e1
        p0 = p0 | (e0 & (c0 == cm))
        p1 = p1 | (e1 & (c1 == cm))
        return d0, d1, p0, p1

    false16 = lanes < 0
    d0, d1, p0, p1 = lax.fori_loop(1, N, dbody,
                                   (false16, false16, false16, false16))
    v0 = lanes < N
    v1 = (lanes + 16) < N
    keep_v[pl.ds(0, 16)] = jnp.where(v0 & (~d0), 1.0, 0.0)
    keep_v[pl.ds(16, 16)] = jnp.where(v1 & (~d1), 1.0, 0.0)
    pk_v[pl.ds(0, 16)] = jnp.where(v0 & (~p0), 1.0, 0.0)
    pk_v[pl.ds(16, 16)] = jnp.where(v1 & (~p1), 1.0, 0.0)

    zero16 = jnp.zeros((16,), jnp.float32)
    bbox_s = jnp.float32(0.0)
    objx_s = jnp.float32(0.0)
    objsp_s = jnp.float32(0.0)
    nobj_s = jnp.float32(0.0)
    clsx_s = jnp.float32(0.0)
    for k in range(2):
        n = lanes + 16 * k
        nv = jnp.minimum(n, N - 1)
        keep = keep_v[pl.ds(16 * k, 16)]
        pk = pk_v[pl.ds(16 * k, 16)]
        rr = r0 + nv
        bacc = zero16
        for j in range(4):
            pj = plsc.load_gather(g_v, [rr, jnp.full((16,), j, jnp.int32)])
            tj = plsc.load_gather(tb_v, [nv * 4 + j])
            d = pj - tj
            bacc = bacc + d * d
        bbox_s = bbox_s + jnp.sum(keep * bacc)
        po = plsc.load_gather(g_v, [rr, jnp.full((16,), 4, jnp.int32)])
        objx_s = objx_s + jnp.sum(keep * po)
        objsp_s = objsp_s + jnp.sum(keep * _softplus16(po))
        nobj_s = nobj_s + jnp.sum(keep)
        cv = tc_v[pl.ds(16 * k, 16)]
        xc = plsc.load_gather(g_v, [rr, 5 + jnp.clip(cv, 0, NCLS - 1)])
        clsx_s = clsx_s + jnp.sum(pk * xc)

    def cbody(c, accs):
        a0, a1 = accs
        for k in range(2):
            n = lanes + 16 * k
            nv = jnp.minimum(n, N - 1)
            keep = keep_v[pl.ds(16 * k, 16)]
            pc = plsc.load_gather(g_v, [r0 + nv, jnp.full((16,), 5, jnp.int32) + c])
            if k == 0:
                a0 = a0 + keep * _softplus16(pc)
            else:
                a1 = a1 + keep * _softplus16(pc)
        return a0, a1

    a0, a1 = lax.fori_loop(0, NCLS, cbody, (zero16, zero16))
    clssp_s = jnp.sum(a0 + a1)
    return bbox_s, objx_s, objsp_s, nobj_s, clssp_s, clsx_s


def _emit_row(lanes, bbox_s, objx_s, objsp_s, nobj_s, clssp_s, clsx_s, HW):
    obj_b = (objx_s + 0.5 * objsp_s) * (1.0 / HW)
    cls_num = clssp_s - clsx_s
    den = jnp.maximum(nobj_s * float(NCLS), 1.0)
    row = jnp.where(lanes == 0, 5.0 * bbox_s, 0.0)
    row = jnp.where(lanes == 1, obj_b, row)
    row = row + jnp.where((lanes == 2) & (nobj_s > 0.0), cls_num, 0.0) / den
    return row


def _sc_partials(d3, d4, d5, tbf, tcf):
    info = plsc.get_sparse_core_info()
    nc = info.num_cores
    mesh = plsc.VectorSubcoreMesh(core_axis_name="c", subcore_axis_name="s")

    @functools.partial(
        pl.kernel,
        out_type=jax.ShapeDtypeStruct((80 * 16,), jnp.float32),
        mesh=mesh,
        compiler_params=pltpu.CompilerParams(needs_layout_passes=False),
        scratch_types=[
            pltpu.VMEM((4, 64, 85), jnp.float32),
            pltpu.VMEM((4, 64, 85), jnp.float32),
            pltpu.VMEM((40, NROW), jnp.float32),
            pltpu.VMEM((80,), jnp.float32),
            pltpu.VMEM((32,), jnp.int32),
            pltpu.VMEM((64,), jnp.int32),
            pltpu.VMEM((32,), jnp.float32),
            pltpu.VMEM((32,), jnp.float32),
            pltpu.VMEM((16,), jnp.float32),
            pltpu.SemaphoreType.DMA,
            pltpu.SemaphoreType.DMA,
            pltpu.SemaphoreType.DMA,
        ],
    )
    def k(d3r, d4r, d5r, tbr, tcr, outr, vb0, vb1, g_v, tb_v, tc_v, flat_v,
          keep_v, pk_v, row_v, gsem, dsem0, dsem1):
        w = lax.axis_index("s") * nc + lax.axis_index("c")
        lanes = lax.iota(jnp.int32, 16)
        b1 = jnp.where(w < 16, w, w - 16)

        pltpu.sync_copy(tbr.at[pl.ds(b1 * 80, 80)], tb_v)
        pltpu.sync_copy(tcr.at[pl.ds(b1 * 32, 32)], tc_v)

        @pl.when(w < 16)
        def _():
            fls = _compute_cells(tb_v, flat_v, 64, 64, 0)
            _fire_cells(d3r, g_v, gsem, fls, b1, 64, 0)
            fls = _compute_cells(tb_v, flat_v, 16, 16, 32)
            _fire_cells(d5r, g_v, gsem, fls, b1, 16, N)

        @pl.when(w >= 16)
        def _():
            fls = _compute_cells(tb_v, flat_v, 32, 32, 0)
            _fire_cells(d4r, g_v, gsem, fls, b1, 32, 0)

        s4 = _dense_scan(d4r, vb0, vb1, dsem0, dsem1, 32, 32, w, 4)
        s5 = _dense_scan(d5r, vb0, vb1, dsem0, dsem1, 16, 16, w, 2)
        drow = jnp.where(lanes == 1, s4, 0.0)
        drow = jnp.where(lanes == 2, s5, drow)
        row_v[...] = drow
        pltpu.sync_copy(row_v, outr.at[pl.ds((48 + w) * 16, 16)])

        @pl.when(w < 16)
        def _():
            for _ in range(2 * N):
                pltpu.make_async_copy(
                    d3r.at[0, 0, pl.ds(0, 1), :],
                    g_v.at[pl.ds(0, 1), :], gsem).wait()
            r = _sparse_task(g_v, tb_v, tc_v, flat_v, keep_v, pk_v, 0, 0)
            row_v[...] = _emit_row(lanes, *r, 4096)
            pltpu.sync_copy(row_v, outr.at[pl.ds(w * 16, 16)])
            r = _sparse_task(g_v, tb_v, tc_v, flat_v, keep_v, pk_v, N, 32)
            row_v[...] = _emit_row(lanes, *r, 256)
            pltpu.sync_copy(row_v, outr.at[pl.ds((w + 32) * 16, 16)])

        @pl.when(w >= 16)
        def _():
            for _ in range(N):
                pltpu.make_async_copy(
                    d4r.at[0, 0, pl.ds(0, 1), :],
                    g_v.at[pl.ds(0, 1), :], gsem).wait()
            r = _sparse_task(g_v, tb_v, tc_v, flat_v, keep_v, pk_v, 0, 0)
            row_v[...] = _emit_row(lanes, *r, 1024)
            pltpu.sync_copy(row_v, outr.at[pl.ds(w * 16, 16)])

    return k(d3, d4, d5, tbf, tcf)


def _p3_dense_kernel(x_ref, o_ref, acc):
    b = pl.program_id(0)
    h = pl.program_id(1)

    @pl.when((b == 0) & (h == 0))
    def _():
        acc[0] = 0.0

    x = x_ref[0]
    obj = x[:, :, 4]
    acc[0] += jnp.sum(jax.nn.softplus(obj))

    @pl.when((b == 15) & (h == 7))
    def _():
        i = lax.broadcasted_iota(jnp.int32, (1, 16), 1)
        o_ref[...] = jnp.where(i == 0, acc[0], 0.0)


def _combine_kernel(p_ref, d3_ref, o_ref):
    p = p_ref[...]
    task = p[0:48, :]
    dense = p[48:80, :]
    lane = lax.broadcasted_iota(jnp.int32, (48, 16), 1)
    dsum = jnp.sum(dense, axis=0, keepdims=True)
    dl = lax.broadcasted_iota(jnp.int32, (1, 16), 1)
    s3 = jnp.sum(jnp.where(dl == 0, d3_ref[...], 0.0))
    s4 = jnp.sum(jnp.where(dl == 1, dsum, 0.0))
    s5 = jnp.sum(jnp.where(dl == 2, dsum, 0.0))
    bbox = jnp.sum(jnp.where(lane == 0, task, 0.0)) / 48.0
    objsp = jnp.sum(jnp.where(lane == 1, task, 0.0))
    obj = (1.5 * (s3 / 4096.0 + s4 / 1024.0 + s5 / 256.0) - objsp) / 48.0
    cls = jnp.sum(jnp.where(lane == 2, task, 0.0)) / 48.0
    tot = bbox + obj + cls
    o = jnp.where(dl == 0, bbox, 0.0)
    o = jnp.where(dl == 1, obj, o)
    o = jnp.where(dl == 2, cls, o)
    o = jnp.where(dl == 3, tot, o)
    o_ref[...] = o


def kernel(det_p3, det_p4, det_p5, targets_box, targets_cls):
    t3 = jnp.transpose(det_p3, (0, 2, 3, 1))
    t4 = jnp.transpose(det_p4, (0, 2, 3, 1))
    t5 = jnp.transpose(det_p5, (0, 2, 3, 1))
    tbf = targets_box.reshape(-1)
    tcf = jnp.pad(targets_cls.astype(jnp.int32), ((0, 0), (0, 32 - N))).reshape(-1)

    partials = _sc_partials(t3, t4, t5, tbf, tcf).reshape(80, 16)

    d3sum = pl.pallas_call(
        _p3_dense_kernel,
        grid=(16, 8),
        in_specs=[pl.BlockSpec((1, 8, 64, 85), lambda b, h: (b, h, 0, 0))],
        out_specs=pl.BlockSpec((1, 16), lambda b, h: (0, 0)),
        out_shape=jax.ShapeDtypeStruct((1, 16), jnp.float32),
        scratch_shapes=[pltpu.SMEM((4,), jnp.float32)],
    )(t3)

    out = pl.pallas_call(
        _combine_kernel,
        out_shape=jax.ShapeDtypeStruct((1, 16), jnp.float32),
    )(partials, d3sum)
    return out[0, :4]


# speedup vs baseline: 1.8517x; 1.8517x over previous
import functools

import jax
import jax.numpy as jnp
from jax import lax
from jax.experimental import pallas as pl
from jax.experimental.pallas import tpu as pltpu
from jax.experimental.pallas import tpu_sc as plsc

B = 16
NCLS = 80
C = NCLS + 5
N = 20
NROW = 85
NW = 32


def _softplus16(v):
    m = jnp.maximum(v, 0.0)
    t = jnp.exp(-jnp.abs(v))
    z = t / (2.0 + t)
    z2 = z * z
    p = 1.0 / 9.0
    p = 1.0 / 7.0 + z2 * p
    p = 1.0 / 5.0 + z2 * p
    p = 1.0 / 3.0 + z2 * p
    p = 1.0 + z2 * p
    return m + 2.0 * z * p


def _compute_cells(tb_v, flat_v, H, W, k0):
    lanes = lax.iota(jnp.int32, 16)
    out = []
    for k in range(2):
        n = lanes + 16 * k
        nv = jnp.minimum(n, N - 1)
        i4 = nv * 4
        x = plsc.load_gather(tb_v, [i4])
        y = plsc.load_gather(tb_v, [i4 + 1])
        gx = jnp.clip((x * float(W)).astype(jnp.int32), 0, W - 1)
        gy = jnp.clip((y * float(H)).astype(jnp.int32), 0, H - 1)
        fl = gy * W + gx
        fl = jnp.where(n < N, fl, -1)
        flat_v[pl.ds(k0 + 16 * k, 16)] = fl
        out.append(fl)
    return out


def _fire_cells(det, g_v, gsem, fls, b, W, r0):
    lanes = lax.iota(jnp.int32, 16)
    shift = W.bit_length() - 1
    for n in range(N):
        fl = fls[n // 16]
        fl_s = jnp.sum(jnp.where(lanes == (n % 16), fl, 0))
        gy = lax.shift_right_logical(fl_s, shift)
        gx = fl_s & (W - 1)
        pltpu.make_async_copy(
            det.at[b, gy, pl.ds(gx, 1), :],
            g_v.at[pl.ds(r0 + n, 1), :],
            gsem).start()


def _dense_scan(det, vb0, vb1, dsem0, dsem1, H, W, w, nper):
    nslab = H // 4
    lanes = lax.iota(jnp.int32, 16)
    bufs = (vb0, vb1)
    sems = (dsem0, dsem1)

    def fire(j, buf, sem):
        u = w * nper + j
        bb = u // nslab
        y0 = (u % nslab) * 4
        pltpu.make_async_copy(det.at[bb, pl.ds(y0, 4), :, :],
                              buf.at[:, pl.ds(0, W), :], sem).start()

    def wait(buf, sem):
        pltpu.make_async_copy(det.at[0, pl.ds(0, 4), :, :],
                              buf.at[:, pl.ds(0, W), :], sem).wait()

    def extract(buf):
        acc = jnp.zeros((16,), jnp.float32)
        c4 = jnp.full((16,), 4, jnp.int32)
        shift = W.bit_length() - 1
        nvec = (4 * W) // 16
        for i in range(nvec):
            cell = lanes + 16 * i
            r = lax.shift_right_logical(cell, shift)
            x = cell & (W - 1)
            acc = acc + _softplus16(plsc.load_gather(buf, [r, x, c4]))
        return jnp.sum(acc)

    fire(0, bufs[0], sems[0])
    s = jnp.float32(0.0)
    for j in range(nper):
        if j + 1 < nper:
            fire(j + 1, bufs[(j + 1) % 2], sems[(j + 1) % 2])
        wait(bufs[j % 2], sems[j % 2])
        s = s + extract(bufs[j % 2])
    return s


def _sparse_task(g_v, tb_v, tc_v, flat_v, keep_v, pk_v, r0, k0):
    lanes = lax.iota(jnp.int32, 16)

    f0 = flat_v[pl.ds(k0, 16)]
    f1 = flat_v[pl.ds(k0 + 16, 16)]
    c0 = tc_v[pl.ds(0, 16)]
    c1 = tc_v[pl.ds(16, 16)]

    def dbody(m, carry):
        d0, d1, p0, p1 = carry
        fm = plsc.load_gather(flat_v, [jnp.full((16,), k0 + m, jnp.int32)])
        cm = plsc.load_gather(tc_v, [jnp.full((16,), m, jnp.int32)])
        e0 = (f0 == fm) & (lanes < m)
        e1 = (f1 == fm) & ((lanes + 16) < m)
        d0 = d0 | e0
        d1 = d1 | e1
        p0 = p0 | (e0 & (c0 == cm))
        p1 = p1 | (e1 & (c1 == cm))
        return d0, d1, p0, p1

    false16 = lanes < 0
    d0, d1, p0, p1 = lax.fori_loop(1, N, dbody,
                                   (false16, false16, false16, false16))
    v0 = lanes < N
    v1 = (lanes + 16) < N
    keep_v[pl.ds(0, 16)] = jnp.where(v0 & (~d0), 1.0, 0.0)
    keep_v[pl.ds(16, 16)] = jnp.where(v1 & (~d1), 1.0, 0.0)
    pk_v[pl.ds(0, 16)] = jnp.where(v0 & (~p0), 1.0, 0.0)
    pk_v[pl.ds(16, 16)] = jnp.where(v1 & (~p1), 1.0, 0.0)

    zero16 = jnp.zeros((16,), jnp.float32)
    bbox_s = jnp.float32(0.0)
    objx_s = jnp.float32(0.0)
    objsp_s = jnp.float32(0.0)
    nobj_s = jnp.float32(0.0)
    clsx_s = jnp.float32(0.0)
    for k in range(2):
        n = lanes + 16 * k
        nv = jnp.minimum(n, N - 1)
        keep = keep_v[pl.ds(16 * k, 16)]
        pk = pk_v[pl.ds(16 * k, 16)]
        rr = r0 + nv
        bacc = zero16
        for j in range(4):
            pj = plsc.load_gather(g_v, [rr, jnp.full((16,), j, jnp.int32)])
            tj = plsc.load_gather(tb_v, [nv * 4 + j])
            d = pj - tj
            bacc = bacc + d * d
        bbox_s = bbox_s + jnp.sum(keep * bacc)
        po = plsc.load_gather(g_v, [rr, jnp.full((16,), 4, jnp.int32)])
        objx_s = objx_s + jnp.sum(keep * po)
        objsp_s = objsp_s + jnp.sum(keep * _softplus16(po))
        nobj_s = nobj_s + jnp.sum(keep)
        cv = tc_v[pl.ds(16 * k, 16)]
        xc = plsc.load_gather(g_v, [rr, 5 + jnp.clip(cv, 0, NCLS - 1)])
        clsx_s = clsx_s + jnp.sum(pk * xc)

    def cbody(c, accs):
        a0, a1 = accs
        for k in range(2):
            n = lanes + 16 * k
            nv = jnp.minimum(n, N - 1)
            keep = keep_v[pl.ds(16 * k, 16)]
            pc = plsc.load_gather(g_v, [r0 + nv, jnp.full((16,), 5, jnp.int32) + c])
            if k == 0:
                a0 = a0 + keep * _softplus16(pc)
            else:
                a1 = a1 + keep * _softplus16(pc)
        return a0, a1

    a0, a1 = lax.fori_loop(0, NCLS, cbody, (zero16, zero16))
    clssp_s = jnp.sum(a0 + a1)
    return bbox_s, objx_s, objsp_s, nobj_s, clssp_s, clsx_s


def _emit_row(lanes, bbox_s, objx_s, objsp_s, nobj_s, clssp_s, clsx_s, HW):
    obj_b = (objx_s + 0.5 * objsp_s) * (1.0 / HW)
    cls_num = clssp_s - clsx_s
    den = jnp.maximum(nobj_s * float(NCLS), 1.0)
    row = jnp.where(lanes == 0, 5.0 * bbox_s, 0.0)
    row = jnp.where(lanes == 1, obj_b, row)
    row = row + jnp.where((lanes == 2) & (nobj_s > 0.0), cls_num, 0.0) / den
    return row


def _sc_partials(d3, d4, d5, tbf, tcf):
    info = plsc.get_sparse_core_info()
    nc = info.num_cores
    mesh = plsc.VectorSubcoreMesh(core_axis_name="c", subcore_axis_name="s")

    @functools.partial(
        pl.kernel,
        out_type=jax.ShapeDtypeStruct((80 * 16,), jnp.float32),
        mesh=mesh,
        compiler_params=pltpu.CompilerParams(needs_layout_passes=False),
        scratch_types=[
            pltpu.VMEM((4, 64, 85), jnp.float32),
            pltpu.VMEM((4, 64, 85), jnp.float32),
            pltpu.VMEM((40, NROW), jnp.float32),
            pltpu.VMEM((80,), jnp.float32),
            pltpu.VMEM((32,), jnp.int32),
            pltpu.VMEM((64,), jnp.int32),
            pltpu.VMEM((32,), jnp.float32),
            pltpu.VMEM((32,), jnp.float32),
            pltpu.VMEM((16,), jnp.float32),
            pltpu.SemaphoreType.DMA,
            pltpu.SemaphoreType.DMA,
            pltpu.SemaphoreType.DMA,
        ],
    )
    def k(d3r, d4r, d5r, tbr, tcr, outr, vb0, vb1, g_v, tb_v, tc_v, flat_v,
          keep_v, pk_v, row_v, gsem, dsem0, dsem1):
        w = lax.axis_index("s") * nc + lax.axis_index("c")
        lanes = lax.iota(jnp.int32, 16)
        b1 = jnp.where(w < 16, w, w - 16)

        pltpu.sync_copy(tbr.at[pl.ds(b1 * 80, 80)], tb_v)
        pltpu.sync_copy(tcr.at[pl.ds(b1 * 32, 32)], tc_v)

        @pl.when(w < 16)
        def _():
            fls = _compute_cells(tb_v, flat_v, 64, 64, 0)
            _fire_cells(d3r, g_v, gsem, fls, b1, 64, 0)
            fls = _compute_cells(tb_v, flat_v, 16, 16, 32)
            _fire_cells(d5r, g_v, gsem, fls, b1, 16, N)

        @pl.when(w >= 16)
        def _():
            fls = _compute_cells(tb_v, flat_v, 32, 32, 0)
            _fire_cells(d4r, g_v, gsem, fls, b1, 32, 0)

        s3 = _dense_scan(d3r, vb0, vb1, dsem0, dsem1, 64, 64, w, 8)
        s4 = _dense_scan(d4r, vb0, vb1, dsem0, dsem1, 32, 32, w, 4)
        s5 = _dense_scan(d5r, vb0, vb1, dsem0, dsem1, 16, 16, w, 2)
        drow = jnp.where(lanes == 0, s3, 0.0)
        drow = jnp.where(lanes == 1, s4, drow)
        drow = jnp.where(lanes == 2, s5, drow)
        row_v[...] = drow
        pltpu.sync_copy(row_v, outr.at[pl.ds((48 + w) * 16, 16)])

        @pl.when(w < 16)
        def _():
            for _ in range(2 * N):
                pltpu.make_async_copy(
                    d3r.at[0, 0, pl.ds(0, 1), :],
                    g_v.at[pl.ds(0, 1), :], gsem).wait()
            r = _sparse_task(g_v, tb_v, tc_v, flat_v, keep_v, pk_v, 0, 0)
            row_v[...] = _emit_row(lanes, *r, 4096)
            pltpu.sync_copy(row_v, outr.at[pl.ds(w * 16, 16)])
            r = _sparse_task(g_v, tb_v, tc_v, flat_v, keep_v, pk_v, N, 32)
            row_v[...] = _emit_row(lanes, *r, 256)
            pltpu.sync_copy(row_v, outr.at[pl.ds((w + 32) * 16, 16)])

        @pl.when(w >= 16)
        def _():
            for _ in range(N):
                pltpu.make_async_copy(
                    d4r.at[0, 0, pl.ds(0, 1), :],
                    g_v.at[pl.ds(0, 1), :], gsem).wait()
            r = _sparse_task(g_v, tb_v, tc_v, flat_v, keep_v, pk_v, 0, 0)
            row_v[...] = _emit_row(lanes, *r, 1024)
            pltpu.sync_copy(row_v, outr.at[pl.ds(w * 16, 16)])

    return k(d3, d4, d5, tbf, tcf)


def _combine_kernel(p_ref, o_ref):
    p = p_ref[...]
    task = p[0:48, :]
    dense = p[48:80, :]
    lane = lax.broadcasted_iota(jnp.int32, (48, 16), 1)
    dsum = jnp.sum(dense, axis=0, keepdims=True)
    dl = lax.broadcasted_iota(jnp.int32, (1, 16), 1)
    s3 = jnp.sum(jnp.where(dl == 0, dsum, 0.0))
    s4 = jnp.sum(jnp.where(dl == 1, dsum, 0.0))
    s5 = jnp.sum(jnp.where(dl == 2, dsum, 0.0))
    bbox = jnp.sum(jnp.where(lane == 0, task, 0.0)) / 48.0
    objsp = jnp.sum(jnp.where(lane == 1, task, 0.0))
    obj = (1.5 * (s3 / 4096.0 + s4 / 1024.0 + s5 / 256.0) - objsp) / 48.0
    cls = jnp.sum(jnp.where(lane == 2, task, 0.0)) / 48.0
    tot = bbox + obj + cls
    o = jnp.where(dl == 0, bbox, 0.0)
    o = jnp.where(dl == 1, obj, o)
    o = jnp.where(dl == 2, cls, o)
    o = jnp.where(dl == 3, tot, o)
    o_ref[...] = o


def kernel(det_p3, det_p4, det_p5, targets_box, targets_cls):
    t3 = jnp.transpose(det_p3, (0, 2, 3, 1))
    t4 = jnp.transpose(det_p4, (0, 2, 3, 1))
    t5 = jnp.transpose(det_p5, (0, 2, 3, 1))
    tbf = targets_box.reshape(-1)
    tcf = jnp.pad(targets_cls.astype(jnp.int32), ((0, 0), (0, 32 - N))).reshape(-1)

    partials = _sc_partials(t3, t4, t5, tbf, tcf).reshape(80, 16)

    out = pl.pallas_call(
        _combine_kernel,
        out_shape=jax.ShapeDtypeStruct((1, 16), jnp.float32),
    )(partials)
    return out[0, :4]


# speedup vs baseline: 2.0417x; 1.1026x over previous
import functools

import jax
import jax.numpy as jnp
from jax import lax
from jax.experimental import pallas as pl
from jax.experimental.pallas import tpu as pltpu
from jax.experimental.pallas import tpu_sc as plsc

B = 16
NCLS = 80
C = NCLS + 5
N = 20
NROW = 85
NW = 32


def _softplus16(v):
    m = jnp.maximum(v, 0.0)
    t = jnp.exp(-jnp.abs(v))
    z = t / (2.0 + t)
    z2 = z * z
    p = 1.0 / 9.0
    p = 1.0 / 7.0 + z2 * p
    p = 1.0 / 5.0 + z2 * p
    p = 1.0 / 3.0 + z2 * p
    p = 1.0 + z2 * p
    return m + 2.0 * z * p


def _compute_cells(tb_v, flat_v, H, W, k0):
    lanes = lax.iota(jnp.int32, 16)
    out = []
    for k in range(2):
        n = lanes + 16 * k
        nv = jnp.minimum(n, N - 1)
        i4 = nv * 4
        x = plsc.load_gather(tb_v, [i4])
        y = plsc.load_gather(tb_v, [i4 + 1])
        gx = jnp.clip((x * float(W)).astype(jnp.int32), 0, W - 1)
        gy = jnp.clip((y * float(H)).astype(jnp.int32), 0, H - 1)
        fl = gy * W + gx
        fl = jnp.where(n < N, fl, -1)
        flat_v[pl.ds(k0 + 16 * k, 16)] = fl
        out.append(fl)
    return out


def _fire_cells(det, g_v, gsem, fls, b, W, r0):
    lanes = lax.iota(jnp.int32, 16)
    shift = W.bit_length() - 1

    def body(n, _):
        fl = jnp.where(n < 16, fls[0], fls[1])
        fl_s = jnp.sum(jnp.where(lanes == (n & 15), fl, 0))
        gy = lax.shift_right_logical(fl_s, shift)
        gx = fl_s & (W - 1)
        pltpu.make_async_copy(
            det.at[b, gy, pl.ds(gx, 1), :],
            g_v.at[pl.ds(r0 + n, 1), :],
            gsem).start()
        return 0

    lax.fori_loop(0, N, body, 0)


def _dense_scan(det, vb0, vb1, dsem0, dsem1, H, W, w, nper):
    nslab = H // 4
    lanes = lax.iota(jnp.int32, 16)
    bufs = (vb0, vb1)
    sems = (dsem0, dsem1)

    def fire(j, buf, sem):
        u = w * nper + j
        bb = u // nslab
        y0 = (u % nslab) * 4
        pltpu.make_async_copy(det.at[bb, pl.ds(y0, 4), :, :],
                              buf.at[:, pl.ds(0, W), :], sem).start()

    def wait(buf, sem):
        pltpu.make_async_copy(det.at[0, pl.ds(0, 4), :, :],
                              buf.at[:, pl.ds(0, W), :], sem).wait()

    def extract(buf):
        c4 = jnp.full((16,), 4, jnp.int32)
        shift = W.bit_length() - 1
        nvec = (4 * W) // 16

        def ibody(i, acc):
            cell = lanes + 16 * i
            r = lax.shift_right_logical(cell, shift)
            x = cell & (W - 1)
            return acc + _softplus16(plsc.load_gather(buf, [r, x, c4]))

        return jnp.sum(lax.fori_loop(0, nvec, ibody,
                                     jnp.zeros((16,), jnp.float32)))

    fire(0, bufs[0], sems[0])

    def jbody(j2, s):
        j = 2 * j2
        fire(j + 1, bufs[1], sems[1])
        wait(bufs[0], sems[0])
        s = s + extract(bufs[0])

        @pl.when(j2 + 1 < nper // 2)
        def _():
            fire(j + 2, bufs[0], sems[0])

        wait(bufs[1], sems[1])
        return s + extract(bufs[1])

    return lax.fori_loop(0, nper // 2, jbody, jnp.float32(0.0))


def _sparse_task(g_v, tb_v, tc_v, flat_v, keep_v, pk_v, r0, k0):
    lanes = lax.iota(jnp.int32, 16)

    f0 = flat_v[pl.ds(k0, 16)]
    f1 = flat_v[pl.ds(k0 + 16, 16)]
    c0 = tc_v[pl.ds(0, 16)]
    c1 = tc_v[pl.ds(16, 16)]

    def dbody(m, carry):
        d0, d1, p0, p1 = carry
        fm = plsc.load_gather(flat_v, [jnp.full((16,), k0 + m, jnp.int32)])
        cm = plsc.load_gather(tc_v, [jnp.full((16,), m, jnp.int32)])
        e0 = (f0 == fm) & (lanes < m)
        e1 = (f1 == fm) & ((lanes + 16) < m)
        d0 = d0 | e0
        d1 = d1 | e1
        p0 = p0 | (e0 & (c0 == cm))
        p1 = p1 | (e1 & (c1 == cm))
        return d0, d1, p0, p1

    false16 = lanes < 0
    d0, d1, p0, p1 = lax.fori_loop(1, N, dbody,
                                   (false16, false16, false16, false16))
    v0 = lanes < N
    v1 = (lanes + 16) < N
    keep_v[pl.ds(0, 16)] = jnp.where(v0 & (~d0), 1.0, 0.0)
    keep_v[pl.ds(16, 16)] = jnp.where(v1 & (~d1), 1.0, 0.0)
    pk_v[pl.ds(0, 16)] = jnp.where(v0 & (~p0), 1.0, 0.0)
    pk_v[pl.ds(16, 16)] = jnp.where(v1 & (~p1), 1.0, 0.0)

    zero16 = jnp.zeros((16,), jnp.float32)
    bbox_s = jnp.float32(0.0)
    objx_s = jnp.float32(0.0)
    objsp_s = jnp.float32(0.0)
    nobj_s = jnp.float32(0.0)
    clsx_s = jnp.float32(0.0)
    for k in range(2):
        n = lanes + 16 * k
        nv = jnp.minimum(n, N - 1)
        keep = keep_v[pl.ds(16 * k, 16)]
        pk = pk_v[pl.ds(16 * k, 16)]
        rr = r0 + nv
        bacc = zero16
        for j in range(4):
            pj = plsc.load_gather(g_v, [rr, jnp.full((16,), j, jnp.int32)])
            tj = plsc.load_gather(tb_v, [nv * 4 + j])
            d = pj - tj
            bacc = bacc + d * d
        bbox_s = bbox_s + jnp.sum(keep * bacc)
        po = plsc.load_gather(g_v, [rr, jnp.full((16,), 4, jnp.int32)])
        objx_s = objx_s + jnp.sum(keep * po)
        objsp_s = objsp_s + jnp.sum(keep * _softplus16(po))
        nobj_s = nobj_s + jnp.sum(keep)
        cv = tc_v[pl.ds(16 * k, 16)]
        xc = plsc.load_gather(g_v, [rr, 5 + jnp.clip(cv, 0, NCLS - 1)])
        clsx_s = clsx_s + jnp.sum(pk * xc)

    def cbody(c, accs):
        a0, a1 = accs
        for k in range(2):
            n = lanes + 16 * k
            nv = jnp.minimum(n, N - 1)
            keep = keep_v[pl.ds(16 * k, 16)]
            pc = plsc.load_gather(g_v, [r0 + nv, jnp.full((16,), 5, jnp.int32) + c])
            if k == 0:
                a0 = a0 + keep * _softplus16(pc)
            else:
                a1 = a1 + keep * _softplus16(pc)
        return a0, a1

    a0, a1 = lax.fori_loop(0, NCLS, cbody, (zero16, zero16))
    clssp_s = jnp.sum(a0 + a1)
    return bbox_s, objx_s, objsp_s, nobj_s, clssp_s, clsx_s


def _emit_row(lanes, bbox_s, objx_s, objsp_s, nobj_s, clssp_s, clsx_s, HW):
    obj_b = (objx_s + 0.5 * objsp_s) * (1.0 / HW)
    cls_num = clssp_s - clsx_s
    den = jnp.maximum(nobj_s * float(NCLS), 1.0)
    row = jnp.where(lanes == 0, 5.0 * bbox_s, 0.0)
    row = jnp.where(lanes == 1, obj_b, row)
    row = row + jnp.where((lanes == 2) & (nobj_s > 0.0), cls_num, 0.0) / den
    return row


def _sc_partials(d3, d4, d5, tbf, tcf):
    info = plsc.get_sparse_core_info()
    nc = info.num_cores
    mesh = plsc.VectorSubcoreMesh(core_axis_name="c", subcore_axis_name="s")

    @functools.partial(
        pl.kernel,
        out_type=jax.ShapeDtypeStruct((80 * 16,), jnp.float32),
        mesh=mesh,
        compiler_params=pltpu.CompilerParams(needs_layout_passes=False),
        scratch_types=[
            pltpu.VMEM((4, 64, 85), jnp.float32),
            pltpu.VMEM((4, 64, 85), jnp.float32),
            pltpu.VMEM((40, NROW), jnp.float32),
            pltpu.VMEM((80,), jnp.float32),
            pltpu.VMEM((32,), jnp.int32),
            pltpu.VMEM((64,), jnp.int32),
            pltpu.VMEM((32,), jnp.float32),
            pltpu.VMEM((32,), jnp.float32),
            pltpu.VMEM((16,), jnp.float32),
            pltpu.SemaphoreType.DMA,
            pltpu.SemaphoreType.DMA,
            pltpu.SemaphoreType.DMA,
        ],
    )
    def k(d3r, d4r, d5r, tbr, tcr, outr, vb0, vb1, g_v, tb_v, tc_v, flat_v,
          keep_v, pk_v, row_v, gsem, dsem0, dsem1):
        w = lax.axis_index("s") * nc + lax.axis_index("c")
        lanes = lax.iota(jnp.int32, 16)
        b1 = jnp.where(w < 16, w, w - 16)

        pltpu.sync_copy(tbr.at[pl.ds(b1 * 80, 80)], tb_v)
        pltpu.sync_copy(tcr.at[pl.ds(b1 * 32, 32)], tc_v)

        @pl.when(w < 16)
        def _():
            fls = _compute_cells(tb_v, flat_v, 64, 64, 0)
            _fire_cells(d3r, g_v, gsem, fls, b1, 64, 0)
            fls = _compute_cells(tb_v, flat_v, 16, 16, 32)
            _fire_cells(d5r, g_v, gsem, fls, b1, 16, N)

        @pl.when(w >= 16)
        def _():
            fls = _compute_cells(tb_v, flat_v, 32, 32, 0)
            _fire_cells(d4r, g_v, gsem, fls, b1, 32, 0)

        s3 = _dense_scan(d3r, vb0, vb1, dsem0, dsem1, 64, 64, w, 8)
        s4 = _dense_scan(d4r, vb0, vb1, dsem0, dsem1, 32, 32, w, 4)
        s5 = _dense_scan(d5r, vb0, vb1, dsem0, dsem1, 16, 16, w, 2)
        drow = jnp.where(lanes == 0, s3, 0.0)
        drow = jnp.where(lanes == 1, s4, drow)
        drow = jnp.where(lanes == 2, s5, drow)
        row_v[...] = drow
        pltpu.sync_copy(row_v, outr.at[pl.ds((48 + w) * 16, 16)])

        @pl.when(w < 16)
        def _():
            for _ in range(2 * N):
                pltpu.make_async_copy(
                    d3r.at[0, 0, pl.ds(0, 1), :],
                    g_v.at[pl.ds(0, 1), :], gsem).wait()
            r = _sparse_task(g_v, tb_v, tc_v, flat_v, keep_v, pk_v, 0, 0)
            row_v[...] = _emit_row(lanes, *r, 4096)
            pltpu.sync_copy(row_v, outr.at[pl.ds(w * 16, 16)])
            r = _sparse_task(g_v, tb_v, tc_v, flat_v, keep_v, pk_v, N, 32)
            row_v[...] = _emit_row(lanes, *r, 256)
            pltpu.sync_copy(row_v, outr.at[pl.ds((w + 32) * 16, 16)])

        @pl.when(w >= 16)
        def _():
            for _ in range(N):
                pltpu.make_async_copy(
                    d4r.at[0, 0, pl.ds(0, 1), :],
                    g_v.at[pl.ds(0, 1), :], gsem).wait()
            r = _sparse_task(g_v, tb_v, tc_v, flat_v, keep_v, pk_v, 0, 0)
            row_v[...] = _emit_row(lanes, *r, 1024)
            pltpu.sync_copy(row_v, outr.at[pl.ds(w * 16, 16)])

    return k(d3, d4, d5, tbf, tcf)


def _combine_kernel(p_ref, o_ref):
    p = p_ref[...]
    task = p[0:48, :]
    dense = p[48:80, :]
    lane = lax.broadcasted_iota(jnp.int32, (48, 16), 1)
    dsum = jnp.sum(dense, axis=0, keepdims=True)
    dl = lax.broadcasted_iota(jnp.int32, (1, 16), 1)
    s3 = jnp.sum(jnp.where(dl == 0, dsum, 0.0))
    s4 = jnp.sum(jnp.where(dl == 1, dsum, 0.0))
    s5 = jnp.sum(jnp.where(dl == 2, dsum, 0.0))
    bbox = jnp.sum(jnp.where(lane == 0, task, 0.0)) / 48.0
    objsp = jnp.sum(jnp.where(lane == 1, task, 0.0))
    obj = (1.5 * (s3 / 4096.0 + s4 / 1024.0 + s5 / 256.0) - objsp) / 48.0
    cls = jnp.sum(jnp.where(lane == 2, task, 0.0)) / 48.0
    tot = bbox + obj + cls
    o = jnp.where(dl == 0, bbox, 0.0)
    o = jnp.where(dl == 1, obj, o)
    o = jnp.where(dl == 2, cls, o)
    o = jnp.where(dl == 3, tot, o)
    o_ref[...] = o


def kernel(det_p3, det_p4, det_p5, targets_box, targets_cls):
    t3 = jnp.transpose(det_p3, (0, 2, 3, 1))
    t4 = jnp.transpose(det_p4, (0, 2, 3, 1))
    t5 = jnp.transpose(det_p5, (0, 2, 3, 1))
    tbf = targets_box.reshape(-1)
    tcf = jnp.pad(targets_cls.astype(jnp.int32), ((0, 0), (0, 32 - N))).reshape(-1)

    partials = _sc_partials(t3, t4, t5, tbf, tcf).reshape(80, 16)

    out = pl.pallas_call(
        _combine_kernel,
        out_shape=jax.ShapeDtypeStruct((1, 16), jnp.float32),
    )(partials)
    return out[0, :4]


# speedup vs baseline: 2.1411x; 1.0487x over previous
import functools

import jax
import jax.numpy as jnp
from jax import lax
from jax.experimental import pallas as pl
from jax.experimental.pallas import tpu as pltpu
from jax.experimental.pallas import tpu_sc as plsc

B = 16
NCLS = 80
C = NCLS + 5
N = 20
NROW = 85
NW = 32


def _softplus16(v):
    m = jnp.maximum(v, 0.0)
    t = jnp.exp(-jnp.abs(v))
    z = t / (2.0 + t)
    z2 = z * z
    p = 1.0 / 9.0
    p = 1.0 / 7.0 + z2 * p
    p = 1.0 / 5.0 + z2 * p
    p = 1.0 / 3.0 + z2 * p
    p = 1.0 + z2 * p
    return m + 2.0 * z * p


def _compute_cells(tb_v, flat_v, H, W, k0):
    lanes = lax.iota(jnp.int32, 16)
    out = []
    for k in range(2):
        n = lanes + 16 * k
        nv = jnp.minimum(n, N - 1)
        i4 = nv * 4
        x = plsc.load_gather(tb_v, [i4])
        y = plsc.load_gather(tb_v, [i4 + 1])
        gx = jnp.clip((x * float(W)).astype(jnp.int32), 0, W - 1)
        gy = jnp.clip((y * float(H)).astype(jnp.int32), 0, H - 1)
        fl = gy * W + gx
        fl = jnp.where(n < N, fl, -1)
        flat_v[pl.ds(k0 + 16 * k, 16)] = fl
        out.append(fl)
    return out


def _fire_cells(det, g_v, gsem, fls, b, W, r0):
    lanes = lax.iota(jnp.int32, 16)
    shift = W.bit_length() - 1

    def body(n, _):
        fl = jnp.where(n < 16, fls[0], fls[1])
        fl_s = jnp.sum(jnp.where(lanes == (n & 15), fl, 0))
        gy = lax.shift_right_logical(fl_s, shift)
        gx = fl_s & (W - 1)
        pltpu.make_async_copy(
            det.at[b, gy, pl.ds(gx, 1), :],
            g_v.at[pl.ds(r0 + n, 1), :],
            gsem).start()
        return 0

    lax.fori_loop(0, N, body, 0)


def _dense_scan(det, vb0, vb1, dsem0, dsem1, H, W, w, nper):
    nslab = H // 4
    lanes = lax.iota(jnp.int32, 16)
    bufs = (vb0, vb1)
    sems = (dsem0, dsem1)

    def fire(j, buf, sem):
        u = w * nper + j
        bb = u // nslab
        y0 = (u % nslab) * 4
        pltpu.make_async_copy(det.at[bb, pl.ds(y0, 4), :, :],
                              buf.at[:, pl.ds(0, W), :], sem).start()

    def wait(buf, sem):
        pltpu.make_async_copy(det.at[0, pl.ds(0, 4), :, :],
                              buf.at[:, pl.ds(0, W), :], sem).wait()

    def extract(buf):
        c4 = jnp.full((16,), 4, jnp.int32)
        shift = W.bit_length() - 1
        nvec = (4 * W) // 16

        def ibody(i, acc):
            cell = lanes + 16 * i
            r = lax.shift_right_logical(cell, shift)
            x = cell & (W - 1)
            return acc + _softplus16(plsc.load_gather(buf, [r, x, c4]))

        return jnp.sum(lax.fori_loop(0, nvec, ibody,
                                     jnp.zeros((16,), jnp.float32)))

    fire(0, bufs[0], sems[0])

    def jbody(j2, s):
        j = 2 * j2
        fire(j + 1, bufs[1], sems[1])
        wait(bufs[0], sems[0])
        s = s + extract(bufs[0])

        @pl.when(j2 + 1 < nper // 2)
        def _():
            fire(j + 2, bufs[0], sems[0])

        wait(bufs[1], sems[1])
        return s + extract(bufs[1])

    return lax.fori_loop(0, nper // 2, jbody, jnp.float32(0.0))


def _sparse_task(g_v, tb_v, tc_v, flat_v, keep_v, pk_v, r0, k0):
    lanes = lax.iota(jnp.int32, 16)

    f0 = flat_v[pl.ds(k0, 16)]
    f1 = flat_v[pl.ds(k0 + 16, 16)]
    c0 = tc_v[pl.ds(0, 16)]
    c1 = tc_v[pl.ds(16, 16)]

    def dbody(m, carry):
        d0, d1, p0, p1 = carry
        fm = plsc.load_gather(flat_v, [jnp.full((16,), k0 + m, jnp.int32)])
        cm = plsc.load_gather(tc_v, [jnp.full((16,), m, jnp.int32)])
        e0 = (f0 == fm) & (lanes < m)
        e1 = (f1 == fm) & ((lanes + 16) < m)
        d0 = d0 | e0
        d1 = d1 | e1
        p0 = p0 | (e0 & (c0 == cm))
        p1 = p1 | (e1 & (c1 == cm))
        return d0, d1, p0, p1

    false16 = lanes < 0
    d0, d1, p0, p1 = lax.fori_loop(1, N, dbody,
                                   (false16, false16, false16, false16))
    v0 = lanes < N
    v1 = (lanes + 16) < N
    keep_v[pl.ds(0, 16)] = jnp.where(v0 & (~d0), 1.0, 0.0)
    keep_v[pl.ds(16, 16)] = jnp.where(v1 & (~d1), 1.0, 0.0)
    pk_v[pl.ds(0, 16)] = jnp.where(v0 & (~p0), 1.0, 0.0)
    pk_v[pl.ds(16, 16)] = jnp.where(v1 & (~p1), 1.0, 0.0)

    zero16 = jnp.zeros((16,), jnp.float32)
    bbox_s = jnp.float32(0.0)
    objx_s = jnp.float32(0.0)
    objsp_s = jnp.float32(0.0)
    nobj_s = jnp.float32(0.0)
    clsx_s = jnp.float32(0.0)
    for k in range(2):
        n = lanes + 16 * k
        nv = jnp.minimum(n, N - 1)
        keep = keep_v[pl.ds(16 * k, 16)]
        pk = pk_v[pl.ds(16 * k, 16)]
        rr = r0 + nv
        bacc = zero16
        for j in range(4):
            pj = plsc.load_gather(g_v, [rr, jnp.full((16,), j, jnp.int32)])
            tj = plsc.load_gather(tb_v, [nv * 4 + j])
            d = pj - tj
            bacc = bacc + d * d
        bbox_s = bbox_s + jnp.sum(keep * bacc)
        po = plsc.load_gather(g_v, [rr, jnp.full((16,), 4, jnp.int32)])
        objx_s = objx_s + jnp.sum(keep * po)
        objsp_s = objsp_s + jnp.sum(keep * _softplus16(po))
        nobj_s = nobj_s + jnp.sum(keep)
        cv = tc_v[pl.ds(16 * k, 16)]
        xc = plsc.load_gather(g_v, [rr, 5 + jnp.clip(cv, 0, NCLS - 1)])
        clsx_s = clsx_s + jnp.sum(pk * xc)

    def cbody(c, accs):
        a0, a1 = accs
        for k in range(2):
            n = lanes + 16 * k
            nv = jnp.minimum(n, N - 1)
            keep = keep_v[pl.ds(16 * k, 16)]
            pc = plsc.load_gather(g_v, [r0 + nv, jnp.full((16,), 5, jnp.int32) + c])
            if k == 0:
                a0 = a0 + keep * _softplus16(pc)
            else:
                a1 = a1 + keep * _softplus16(pc)
        return a0, a1

    a0, a1 = lax.fori_loop(0, NCLS, cbody, (zero16, zero16))
    clssp_s = jnp.sum(a0 + a1)
    return bbox_s, objx_s, objsp_s, nobj_s, clssp_s, clsx_s


def _contrib(lanes, r, HW):
    bbox_s, objx_s, objsp_s, nobj_s, clssp_s, clsx_s = r
    inv = 1.0 / 48.0
    den = jnp.maximum(nobj_s * float(NCLS), 1.0)
    row = jnp.where(lanes == 0, 5.0 * bbox_s * inv, 0.0)
    row = row - jnp.where(lanes == 1,
                          (objx_s + 0.5 * objsp_s) * (inv / HW), 0.0)
    row = row + jnp.where((lanes == 2) & (nobj_s > 0.0),
                          (clssp_s - clsx_s) * inv, 0.0) / den
    return row


def _sc_partials(d3, d4, d5, tbf, tcf):
    info = plsc.get_sparse_core_info()
    nc = info.num_cores
    mesh = plsc.VectorSubcoreMesh(core_axis_name="c", subcore_axis_name="s")

    @functools.partial(
        pl.kernel,
        out_type=jax.ShapeDtypeStruct((32 * 16,), jnp.float32),
        mesh=mesh,
        compiler_params=pltpu.CompilerParams(needs_layout_passes=False),
        scratch_types=[
            pltpu.VMEM((4, 64, 85), jnp.float32),
            pltpu.VMEM((4, 64, 85), jnp.float32),
            pltpu.VMEM((40, NROW), jnp.float32),
            pltpu.VMEM((80,), jnp.float32),
            pltpu.VMEM((32,), jnp.int32),
            pltpu.VMEM((64,), jnp.int32),
            pltpu.VMEM((32,), jnp.float32),
            pltpu.VMEM((32,), jnp.float32),
            pltpu.VMEM((16,), jnp.float32),
            pltpu.SemaphoreType.DMA,
            pltpu.SemaphoreType.DMA,
            pltpu.SemaphoreType.DMA,
        ],
    )
    def k(d3r, d4r, d5r, tbr, tcr, outr, vb0, vb1, g_v, tb_v, tc_v,
          flat_v, keep_v, pk_v, row_v, gsem, dsem0, dsem1):
        sid = lax.axis_index("s")
        cid = lax.axis_index("c")
        w = sid * nc + cid
        lanes = lax.iota(jnp.int32, 16)
        b1 = jnp.where(w < 16, w, w - 16)

        pltpu.sync_copy(tbr.at[pl.ds(b1 * 80, 80)], tb_v)
        pltpu.sync_copy(tcr.at[pl.ds(b1 * 32, 32)], tc_v)

        @pl.when(w < 16)
        def _():
            fls = _compute_cells(tb_v, flat_v, 64, 64, 0)
            _fire_cells(d3r, g_v, gsem, fls, b1, 64, 0)
            fls = _compute_cells(tb_v, flat_v, 16, 16, 32)
            _fire_cells(d5r, g_v, gsem, fls, b1, 16, N)

        @pl.when(w >= 16)
        def _():
            fls = _compute_cells(tb_v, flat_v, 32, 32, 0)
            _fire_cells(d4r, g_v, gsem, fls, b1, 32, 0)

        s3 = _dense_scan(d3r, vb0, vb1, dsem0, dsem1, 64, 64, w, 8)
        s4 = _dense_scan(d4r, vb0, vb1, dsem0, dsem1, 32, 32, w, 4)
        s5 = _dense_scan(d5r, vb0, vb1, dsem0, dsem1, 16, 16, w, 2)
        dense_part = (1.5 / 48.0) * (s3 * (1.0 / 4096.0)
                                     + s4 * (1.0 / 1024.0)
                                     + s5 * (1.0 / 256.0))
        v = jnp.where(lanes == 1, dense_part, 0.0)

        @pl.when(w < 16)
        def _():
            def drain(i, _):
                pltpu.make_async_copy(
                    d3r.at[0, 0, pl.ds(0, 1), :],
                    g_v.at[pl.ds(0, 1), :], gsem).wait()
                return 0
            lax.fori_loop(0, 2 * N, drain, 0)
            r = _sparse_task(g_v, tb_v, tc_v, flat_v, keep_v, pk_v, 0, 0)
            c3 = _contrib(lanes, r, 4096)
            r = _sparse_task(g_v, tb_v, tc_v, flat_v, keep_v, pk_v, N, 32)
            row_v[...] = v + c3 + _contrib(lanes, r, 256)

        @pl.when(w >= 16)
        def _():
            def drain(i, _):
                pltpu.make_async_copy(
                    d4r.at[0, 0, pl.ds(0, 1), :],
                    g_v.at[pl.ds(0, 1), :], gsem).wait()
                return 0
            lax.fori_loop(0, N, drain, 0)
            r = _sparse_task(g_v, tb_v, tc_v, flat_v, keep_v, pk_v, 0, 0)
            row_v[...] = v + _contrib(lanes, r, 1024)

        vrow = row_v[...]
        tot = jnp.sum(jnp.where(lanes < 3, vrow, 0.0))
        row_v[...] = vrow + jnp.where(lanes == 3, tot, 0.0)
        pltpu.sync_copy(row_v, outr.at[pl.ds(w * 16, 16)])

    return k(d3, d4, d5, tbf, tcf)


def kernel(det_p3, det_p4, det_p5, targets_box, targets_cls):
    t3 = jnp.transpose(det_p3, (0, 2, 3, 1))
    t4 = jnp.transpose(det_p4, (0, 2, 3, 1))
    t5 = jnp.transpose(det_p5, (0, 2, 3, 1))

    tbf = targets_box.reshape(-1)
    tcf = jnp.pad(targets_cls.astype(jnp.int32),
                  ((0, 0), (0, 32 - N))).reshape(-1)
    partials = _sc_partials(t3, t4, t5, tbf, tcf).reshape(32, 16)
    return partials.sum(axis=0)[:4]


# speedup vs baseline: 2.2754x; 1.0627x over previous
import functools

import jax
import jax.numpy as jnp
from jax import lax
from jax.experimental import pallas as pl
from jax.experimental.pallas import tpu as pltpu
from jax.experimental.pallas import tpu_sc as plsc

B = 16
NCLS = 80
C = NCLS + 5
N = 20
NROW = 85
NW = 32


def _softplus16(v):
    m = jnp.maximum(v, 0.0)
    t = jnp.exp(-jnp.abs(v))
    z = t / (2.0 + t)
    z2 = z * z
    p = 1.0 / 9.0
    p = 1.0 / 7.0 + z2 * p
    p = 1.0 / 5.0 + z2 * p
    p = 1.0 / 3.0 + z2 * p
    p = 1.0 + z2 * p
    return m + 2.0 * z * p


def _compute_cells(tb_v, flat_v, H, W, k0):
    lanes = lax.iota(jnp.int32, 16)
    out = []
    for k in range(2):
        n = lanes + 16 * k
        nv = jnp.minimum(n, N - 1)
        i4 = nv * 4
        x = plsc.load_gather(tb_v, [i4])
        y = plsc.load_gather(tb_v, [i4 + 1])
        gx = jnp.clip((x * float(W)).astype(jnp.int32), 0, W - 1)
        gy = jnp.clip((y * float(H)).astype(jnp.int32), 0, H - 1)
        fl = gy * W + gx
        fl = jnp.where(n < N, fl, -1)
        flat_v[pl.ds(k0 + 16 * k, 16)] = fl
        out.append(fl)
    return out


def _fire_cells(det, g_v, gsem, fls, b, W, r0):
    lanes = lax.iota(jnp.int32, 16)
    shift = W.bit_length() - 1

    def body(n, _):
        fl = jnp.where(n < 16, fls[0], fls[1])
        fl_s = jnp.sum(jnp.where(lanes == (n & 15), fl, 0))
        gy = lax.shift_right_logical(fl_s, shift)
        gx = fl_s & (W - 1)
        pltpu.make_async_copy(
            det.at[b, gy, pl.ds(gx, 1), :],
            g_v.at[pl.ds(r0 + n, 1), :],
            gsem).start()
        return 0

    lax.fori_loop(0, N, body, 0)


def _dense_scan(det, vb0, vb1, dsem0, dsem1, H, W, w, nper, prefired=False):
    nslab = H // 4
    lanes = lax.iota(jnp.int32, 16)
    bufs = (vb0, vb1)
    sems = (dsem0, dsem1)

    def fire(j, buf, sem):
        u = w * nper + j
        bb = u // nslab
        y0 = (u % nslab) * 4
        pltpu.make_async_copy(det.at[bb, pl.ds(y0, 4), :, :],
                              buf.at[:, pl.ds(0, W), :], sem).start()

    def wait(buf, sem):
        pltpu.make_async_copy(det.at[0, pl.ds(0, 4), :, :],
                              buf.at[:, pl.ds(0, W), :], sem).wait()

    def extract(buf):
        c4 = jnp.full((16,), 4, jnp.int32)
        shift = W.bit_length() - 1
        nvec = (4 * W) // 16

        def ibody(i, acc):
            cell = lanes + 16 * i
            r = lax.shift_right_logical(cell, shift)
            x = cell & (W - 1)
            return acc + _softplus16(plsc.load_gather(buf, [r, x, c4]))

        return jnp.sum(lax.fori_loop(0, nvec, ibody,
                                     jnp.zeros((16,), jnp.float32)))

    if not prefired:
        fire(0, bufs[0], sems[0])

    def jbody(j2, s):
        j = 2 * j2
        fire(j + 1, bufs[1], sems[1])
        wait(bufs[0], sems[0])
        s = s + extract(bufs[0])

        @pl.when(j2 + 1 < nper // 2)
        def _():
            fire(j + 2, bufs[0], sems[0])

        wait(bufs[1], sems[1])
        return s + extract(bufs[1])

    return lax.fori_loop(0, nper // 2, jbody, jnp.float32(0.0))


def _sparse_task(g_v, tb_v, tc_v, flat_v, keep_v, pk_v, r0, k0):
    lanes = lax.iota(jnp.int32, 16)

    f0 = flat_v[pl.ds(k0, 16)]
    f1 = flat_v[pl.ds(k0 + 16, 16)]
    c0 = tc_v[pl.ds(0, 16)]
    c1 = tc_v[pl.ds(16, 16)]

    def dbody(m, carry):
        d0, d1, p0, p1 = carry
        fm = plsc.load_gather(flat_v, [jnp.full((16,), k0 + m, jnp.int32)])
        cm = plsc.load_gather(tc_v, [jnp.full((16,), m, jnp.int32)])
        e0 = (f0 == fm) & (lanes < m)
        e1 = (f1 == fm) & ((lanes + 16) < m)
        d0 = d0 | e0
        d1 = d1 | e1
        p0 = p0 | (e0 & (c0 == cm))
        p1 = p1 | (e1 & (c1 == cm))
        return d0, d1, p0, p1

    false16 = lanes < 0
    d0, d1, p0, p1 = lax.fori_loop(1, N, dbody,
                                   (false16, false16, false16, false16))
    v0 = lanes < N
    v1 = (lanes + 16) < N
    keep_v[pl.ds(0, 16)] = jnp.where(v0 & (~d0), 1.0, 0.0)
    keep_v[pl.ds(16, 16)] = jnp.where(v1 & (~d1), 1.0, 0.0)
    pk_v[pl.ds(0, 16)] = jnp.where(v0 & (~p0), 1.0, 0.0)
    pk_v[pl.ds(16, 16)] = jnp.where(v1 & (~p1), 1.0, 0.0)

    zero16 = jnp.zeros((16,), jnp.float32)
    bbox_s = jnp.float32(0.0)
    objx_s = jnp.float32(0.0)
    objsp_s = jnp.float32(0.0)
    nobj_s = jnp.float32(0.0)
    clsx_s = jnp.float32(0.0)
    for k in range(2):
        n = lanes + 16 * k
        nv = jnp.minimum(n, N - 1)
        keep = keep_v[pl.ds(16 * k, 16)]
        pk = pk_v[pl.ds(16 * k, 16)]
        rr = r0 + nv
        bacc = zero16
        for j in range(4):
            pj = plsc.load_gather(g_v, [rr, jnp.full((16,), j, jnp.int32)])
            tj = plsc.load_gather(tb_v, [nv * 4 + j])
            d = pj - tj
            bacc = bacc + d * d
        bbox_s = bbox_s + jnp.sum(keep * bacc)
        po = plsc.load_gather(g_v, [rr, jnp.full((16,), 4, jnp.int32)])
        objx_s = objx_s + jnp.sum(keep * po)
        objsp_s = objsp_s + jnp.sum(keep * _softplus16(po))
        nobj_s = nobj_s + jnp.sum(keep)
        cv = tc_v[pl.ds(16 * k, 16)]
        xc = plsc.load_gather(g_v, [rr, 5 + jnp.clip(cv, 0, NCLS - 1)])
        clsx_s = clsx_s + jnp.sum(pk * xc)

    def cbody(c, accs):
        a0, a1 = accs
        for k in range(2):
            n = lanes + 16 * k
            nv = jnp.minimum(n, N - 1)
            keep = keep_v[pl.ds(16 * k, 16)]
            pc = plsc.load_gather(g_v, [r0 + nv, jnp.full((16,), 5, jnp.int32) + c])
            if k == 0:
                a0 = a0 + keep * _softplus16(pc)
            else:
                a1 = a1 + keep * _softplus16(pc)
        return a0, a1

    a0, a1 = lax.fori_loop(0, NCLS, cbody, (zero16, zero16))
    clssp_s = jnp.sum(a0 + a1)
    return bbox_s, objx_s, objsp_s, nobj_s, clssp_s, clsx_s


def _contrib(lanes, r, HW):
    bbox_s, objx_s, objsp_s, nobj_s, clssp_s, clsx_s = r
    inv = 1.0 / 48.0
    den = jnp.maximum(nobj_s * float(NCLS), 1.0)
    row = jnp.where(lanes == 0, 5.0 * bbox_s * inv, 0.0)
    row = row - jnp.where(lanes == 1,
                          (objx_s + 0.5 * objsp_s) * (inv / HW), 0.0)
    row = row + jnp.where((lanes == 2) & (nobj_s > 0.0),
                          (clssp_s - clsx_s) * inv, 0.0) / den
    return row


def _sc_partials(d3, d4, d5, tbf, tcf):
    info = plsc.get_sparse_core_info()
    nc = info.num_cores
    mesh = plsc.VectorSubcoreMesh(core_axis_name="c", subcore_axis_name="s")

    @functools.partial(
        pl.kernel,
        out_type=jax.ShapeDtypeStruct((32 * 16,), jnp.float32),
        mesh=mesh,
        compiler_params=pltpu.CompilerParams(needs_layout_passes=False),
        scratch_types=[
            pltpu.VMEM((4, 64, 85), jnp.float32),
            pltpu.VMEM((4, 64, 85), jnp.float32),
            pltpu.VMEM((40, NROW), jnp.float32),
            pltpu.VMEM((80,), jnp.float32),
            pltpu.VMEM((32,), jnp.int32),
            pltpu.VMEM((64,), jnp.int32),
            pltpu.VMEM((32,), jnp.float32),
            pltpu.VMEM((32,), jnp.float32),
            pltpu.VMEM((16,), jnp.float32),
            pltpu.SemaphoreType.DMA,
            pltpu.SemaphoreType.DMA,
            pltpu.SemaphoreType.DMA,
        ],
    )
    def k(d3r, d4r, d5r, tbr, tcr, outr, vb0, vb1, g_v, tb_v, tc_v,
          flat_v, keep_v, pk_v, row_v, gsem, dsem0, dsem1):
        sid = lax.axis_index("s")
        cid = lax.axis_index("c")
        w = sid * nc + cid
        lanes = lax.iota(jnp.int32, 16)
        b1 = jnp.where(w < 16, w, w - 16)

        u0 = w * 8
        pltpu.make_async_copy(
            d3r.at[u0 // 16, pl.ds((u0 % 16) * 4, 4), :, :],
            vb0.at[:, pl.ds(0, 64), :], dsem0).start()

        pltpu.sync_copy(tbr.at[pl.ds(b1 * 80, 80)], tb_v)
        pltpu.sync_copy(tcr.at[pl.ds(b1 * 32, 32)], tc_v)

        @pl.when(w < 16)
        def _():
            fls = _compute_cells(tb_v, flat_v, 64, 64, 0)
            _fire_cells(d3r, g_v, gsem, fls, b1, 64, 0)
            fls = _compute_cells(tb_v, flat_v, 16, 16, 32)
            _fire_cells(d5r, g_v, gsem, fls, b1, 16, N)

        @pl.when(w >= 16)
        def _():
            fls = _compute_cells(tb_v, flat_v, 32, 32, 0)
            _fire_cells(d4r, g_v, gsem, fls, b1, 32, 0)

        s3 = _dense_scan(d3r, vb0, vb1, dsem0, dsem1, 64, 64, w, 8,
                         prefired=True)
        s4 = _dense_scan(d4r, vb0, vb1, dsem0, dsem1, 32, 32, w, 4)
        s5 = _dense_scan(d5r, vb0, vb1, dsem0, dsem1, 16, 16, w, 2)
        dense_part = (1.5 / 48.0) * (s3 * (1.0 / 4096.0)
                                     + s4 * (1.0 / 1024.0)
                                     + s5 * (1.0 / 256.0))
        v = jnp.where(lanes == 1, dense_part, 0.0)

        @pl.when(w < 16)
        def _():
            def drain(i, _):
                pltpu.make_async_copy(
                    d3r.at[0, 0, pl.ds(0, 1), :],
                    g_v.at[pl.ds(0, 1), :], gsem).wait()
                return 0
            lax.fori_loop(0, 2 * N, drain, 0)
            r = _sparse_task(g_v, tb_v, tc_v, flat_v, keep_v, pk_v, 0, 0)
            c3 = _contrib(lanes, r, 4096)
            r = _sparse_task(g_v, tb_v, tc_v, flat_v, keep_v, pk_v, N, 32)
            row_v[...] = v + c3 + _contrib(lanes, r, 256)

        @pl.when(w >= 16)
        def _():
            def drain(i, _):
                pltpu.make_async_copy(
                    d4r.at[0, 0, pl.ds(0, 1), :],
                    g_v.at[pl.ds(0, 1), :], gsem).wait()
                return 0
            lax.fori_loop(0, N, drain, 0)
            r = _sparse_task(g_v, tb_v, tc_v, flat_v, keep_v, pk_v, 0, 0)
            row_v[...] = v + _contrib(lanes, r, 1024)

        vrow = row_v[...]
        tot = jnp.sum(jnp.where(lanes < 3, vrow, 0.0))
        row_v[...] = vrow + jnp.where(lanes == 3, tot, 0.0)
        pltpu.sync_copy(row_v, outr.at[pl.ds(w * 16, 16)])

    return k(d3, d4, d5, tbf, tcf)


def kernel(det_p3, det_p4, det_p5, targets_box, targets_cls):
    t3 = jnp.transpose(det_p3, (0, 2, 3, 1))
    t4 = jnp.transpose(det_p4, (0, 2, 3, 1))
    t5 = jnp.transpose(det_p5, (0, 2, 3, 1))

    tbf = targets_box.reshape(-1)
    tcf = jnp.pad(targets_cls.astype(jnp.int32),
                  ((0, 0), (0, 32 - N))).reshape(-1)
    partials = _sc_partials(t3, t4, t5, tbf, tcf).reshape(32, 16)
    return partials.sum(axis=0)[:4]
